# Initial kernel scaffold; baseline (speedup 1.0000x reference)
import functools

import jax
import jax.numpy as jnp
from jax import lax
from jax.experimental import pallas as pl
from jax.experimental.pallas import tpu as pltpu
from jax.experimental.pallas import tpu_sc as plsc

N = 10000
D = 128
NC = 2
NS = 16
NW = NC * NS
BLK = 128
NBLK = 80
EPW = BLK * NBLK
EPAD = NW * EPW
NP = 10240
RPS = NP // NS
DEG_W = 16
MB = NP // 8

_mesh = plsc.VectorSubcoreMesh(core_axis_name="c", subcore_axis_name="s")


@functools.partial(
    pl.kernel,
    out_type=jax.ShapeDtypeStruct((NC, NP, DEG_W), jnp.float32),
    mesh=_mesh,
    scratch_types=[
        pltpu.VMEM_SHARED((NP, DEG_W), jnp.float32),
        pltpu.VMEM((NBLK, BLK), jnp.int32),
        pltpu.VMEM((BLK, DEG_W), jnp.float32),
    ],
)
def _deg_kernel(dst_hbm, ones_hbm, zeros_hbm, out_hbm, acc_sh, idx_v, ones_v):
    c = lax.axis_index("c")
    s = lax.axis_index("s")
    wid = s * NC + c
    pltpu.sync_copy(dst_hbm.at[pl.ds(wid * NBLK, NBLK)], idx_v)
    pltpu.sync_copy(ones_hbm, ones_v)
    pltpu.sync_copy(zeros_hbm, acc_sh.at[pl.ds(s * RPS, RPS)])
    plsc.subcore_barrier()

    @pl.loop(0, NBLK)
    def _(j):
        pltpu.sync_copy(ones_v, acc_sh.at[idx_v.at[j]], add=True)

    plsc.subcore_barrier()
    pltpu.sync_copy(acc_sh.at[pl.ds(s * RPS, RPS)],
                    out_hbm.at[c, pl.ds(s * RPS, RPS)])


@functools.partial(
    pl.kernel,
    out_type=jax.ShapeDtypeStruct((NC, NP, D), jnp.float32),
    mesh=_mesh,
    scratch_types=[
        pltpu.VMEM_SHARED((NP, D), jnp.float32),
        pltpu.VMEM((NBLK, BLK), jnp.int32),
        pltpu.VMEM((NBLK, BLK), jnp.int32),
        pltpu.VMEM((BLK, D), jnp.float32),
        pltpu.VMEM((BLK, D), jnp.float32),
        pltpu.SemaphoreType.DMA,
        pltpu.SemaphoreType.DMA,
    ],
)
def _scatter_kernel(src_hbm, dst_hbm, table_hbm, zeros_hbm, out_hbm,
                    acc_sh, src_v, dst_v, rows0, rows1, sem0, sem1):
    c = lax.axis_index("c")
    s = lax.axis_index("s")
    wid = s * NC + c
    pltpu.sync_copy(src_hbm.at[pl.ds(wid * NBLK, NBLK)], src_v)
    pltpu.sync_copy(dst_hbm.at[pl.ds(wid * NBLK, NBLK)], dst_v)
    pltpu.sync_copy(zeros_hbm, acc_sh.at[pl.ds(s * RPS, RPS)])
    plsc.subcore_barrier()

    def gather(j, buf, sem):
        pltpu.make_async_copy(table_hbm.at[src_v.at[j]], buf, sem).start()

    def gwait(buf, sem):
        pltpu.make_async_copy(table_hbm.at[src_v.at[0]], buf, sem).wait()

    gather(0, rows0, sem0)
    gather(1, rows1, sem1)

    @pl.loop(0, NBLK - 2, step=2)
    def _(j):
        gwait(rows0, sem0)
        pltpu.sync_copy(rows0, acc_sh.at[dst_v.at[j]], add=True)
        gather(j + 2, rows0, sem0)
        gwait(rows1, sem1)
        pltpu.sync_copy(rows1, acc_sh.at[dst_v.at[j + 1]], add=True)
        gather(j + 3, rows1, sem1)

    gwait(rows0, sem0)
    pltpu.sync_copy(rows0, acc_sh.at[dst_v.at[NBLK - 2]], add=True)
    gwait(rows1, sem1)
    pltpu.sync_copy(rows1, acc_sh.at[dst_v.at[NBLK - 1]], add=True)

    plsc.subcore_barrier()
    pltpu.sync_copy(acc_sh.at[pl.ds(s * RPS, RPS)],
                    out_hbm.at[c, pl.ds(s * RPS, RPS)])


def _mm_body(x_ref, w_ref, o_ref):
    o_ref[...] = jnp.dot(x_ref[...], w_ref[...],
                         preferred_element_type=jnp.float32)


def _scale_body(xw_ref, deg_ref, o_ref):
    deg = deg_ref[0, :, 0] + deg_ref[1, :, 0] + 1.0
    dis = lax.rsqrt(deg)
    o_ref[...] = xw_ref[...] * dis[:, None]


def _final_body(acc_ref, xw2_ref, deg_ref, b1_ref, w2_ref, b2_ref, o_ref):
    deg = deg_ref[0, :, 0] + deg_ref[1, :, 0] + 1.0
    dis = lax.rsqrt(deg)
    t = (acc_ref[0] + acc_ref[1] + xw2_ref[...]) * dis[:, None] + b1_ref[...]
    h = jnp.maximum(t, 0.0)
    o_ref[...] = jnp.dot(h, w2_ref[...],
                         preferred_element_type=jnp.float32) + b2_ref[...]


def kernel(x, edge_index, W1, b1, W2, b2):
    E = edge_index.shape[1]
    n_pad = EPAD - E
    pad_idx = N + (jnp.arange(n_pad, dtype=jnp.int32) % (NP - N))
    src = jnp.concatenate([edge_index[0], pad_idx]).reshape(NW * NBLK, BLK)
    dst = jnp.concatenate([edge_index[1], pad_idx]).reshape(NW * NBLK, BLK)

    x_pad = jnp.pad(x, ((0, NP - N), (0, 0)))
    ones16 = jnp.ones((BLK, DEG_W), jnp.float32)
    zeros16 = jnp.zeros((RPS, DEG_W), jnp.float32)
    zeros128 = jnp.zeros((RPS, D), jnp.float32)
    b1r = b1.reshape(1, D)
    b2r = b2.reshape(1, D)

    degs = _deg_kernel(dst, ones16, zeros16)
    xw = pl.pallas_call(
        _mm_body,
        grid=(NP // MB,),
        in_specs=[pl.BlockSpec((MB, D), lambda i: (i, 0)),
                  pl.BlockSpec((D, D), lambda i: (0, 0))],
        out_specs=pl.BlockSpec((MB, D), lambda i: (i, 0)),
        out_shape=jax.ShapeDtypeStruct((NP, D), jnp.float32),
    )(x_pad, W1)

    xw2 = pl.pallas_call(
        _scale_body,
        grid=(NP // MB,),
        in_specs=[pl.BlockSpec((MB, D), lambda i: (i, 0)),
                  pl.BlockSpec((NC, MB, DEG_W), lambda i: (0, i, 0))],
        out_specs=pl.BlockSpec((MB, D), lambda i: (i, 0)),
        out_shape=jax.ShapeDtypeStruct((NP, D), jnp.float32),
    )(xw, degs)

    accs = _scatter_kernel(src, dst, xw2, zeros128)

    out = pl.pallas_call(
        _final_body,
        grid=(NP // MB,),
        in_specs=[pl.BlockSpec((NC, MB, D), lambda i: (0, i, 0)),
                  pl.BlockSpec((MB, D), lambda i: (i, 0)),
                  pl.BlockSpec((NC, MB, DEG_W), lambda i: (0, i, 0)),
                  pl.BlockSpec((1, D), lambda i: (0, 0)),
                  pl.BlockSpec((D, D), lambda i: (0, 0)),
                  pl.BlockSpec((1, D), lambda i: (0, 0))],
        out_specs=pl.BlockSpec((MB, D), lambda i: (i, 0)),
        out_shape=jax.ShapeDtypeStruct((NP, D), jnp.float32),
    )(accs, xw2, degs, b1r, W2, b2r)

    return out[:N]


# speedup vs baseline: 31.4537x; 31.4537x over previous
import functools

import jax
import jax.numpy as jnp
from jax import lax
from jax.experimental import pallas as pl
from jax.experimental.pallas import tpu as pltpu
from jax.experimental.pallas import tpu_sc as plsc

N = 10000
D = 128
NC = 2
NS = 16
NW = NC * NS
BLK = 128
NBLK = 80
HB = NBLK // 2
EPW = BLK * NBLK
EPAD = NW * EPW
NP = 10240
RPS = NP // NS
DEG_W = 16
MB = NP // 8

_mesh = plsc.VectorSubcoreMesh(core_axis_name="c", subcore_axis_name="s")


@functools.partial(
    pl.kernel,
    out_type=jax.ShapeDtypeStruct((NC, NP, D), jnp.float32),
    mesh=_mesh,
    scratch_types=[
        pltpu.VMEM_SHARED((NP, D), jnp.float32),
        pltpu.VMEM((NBLK, BLK), jnp.int32),
        pltpu.VMEM((BLK, D), jnp.float32),
    ],
)
def _deg_kernel(dst_hbm, ones_hbm, zeros_hbm, out_hbm, acc_sh, idx_v, buf):
    c = lax.axis_index("c")
    s = lax.axis_index("s")
    wid = s * NC + c
    pltpu.sync_copy(dst_hbm.at[pl.ds(wid * NBLK, NBLK)], idx_v)
    pltpu.sync_copy(zeros_hbm, buf)

    @pl.loop(0, RPS // BLK)
    def _(k):
        pltpu.sync_copy(buf, acc_sh.at[pl.ds(s * RPS + k * BLK, BLK)])

    pltpu.sync_copy(ones_hbm, buf)
    plsc.subcore_barrier()

    @pl.loop(0, NBLK)
    def _(j):
        pltpu.sync_copy(buf, acc_sh.at[idx_v.at[j]], add=True)

    plsc.subcore_barrier()

    @pl.loop(0, RPS // BLK)
    def _(k):
        pltpu.sync_copy(acc_sh.at[pl.ds(s * RPS + k * BLK, BLK)], buf)
        pltpu.sync_copy(buf, out_hbm.at[c, pl.ds(s * RPS + k * BLK, BLK)])


@functools.partial(
    pl.kernel,
    out_type=jax.ShapeDtypeStruct((NC, NP, D), jnp.float32),
    mesh=_mesh,
    scratch_types=[
        pltpu.VMEM_SHARED((NP, D), jnp.float32),
        pltpu.VMEM((HB, BLK), jnp.int32),
        pltpu.VMEM((HB, BLK), jnp.int32),
        pltpu.VMEM((BLK, D), jnp.float32),
        pltpu.VMEM((BLK, D), jnp.float32),
        pltpu.SemaphoreType.DMA,
        pltpu.SemaphoreType.DMA,
    ],
)
def _scatter_kernel(src_hbm, dst_hbm, table_hbm, out_hbm,
                    acc_sh, src_v, dst_v, rows0, rows1, sem0, sem1):
    c = lax.axis_index("c")
    s = lax.axis_index("s")
    wid = s * NC + c
    pltpu.sync_copy(table_hbm.at[pl.ds(N, BLK)], rows0)

    @pl.loop(0, RPS // BLK)
    def _(k):
        pltpu.sync_copy(rows0, acc_sh.at[pl.ds(s * RPS + k * BLK, BLK)])

    plsc.subcore_barrier()

    def gather(j, buf, sem):
        pltpu.make_async_copy(table_hbm.at[src_v.at[j]], buf, sem).start()

    def gwait(buf, sem):
        pltpu.make_async_copy(table_hbm.at[src_v.at[0]], buf, sem).wait()

    @pl.loop(0, 2)
    def _(h):
        base = wid * NBLK + h * HB
        pltpu.sync_copy(src_hbm.at[pl.ds(base, HB)], src_v)
        pltpu.sync_copy(dst_hbm.at[pl.ds(base, HB)], dst_v)

        gather(0, rows0, sem0)
        gather(1, rows1, sem1)

        @pl.loop(0, HB - 2, step=2)
        def _(j):
            gwait(rows0, sem0)
            pltpu.sync_copy(rows0, acc_sh.at[dst_v.at[j]], add=True)
            gather(j + 2, rows0, sem0)
            gwait(rows1, sem1)
            pltpu.sync_copy(rows1, acc_sh.at[dst_v.at[j + 1]], add=True)
            gather(j + 3, rows1, sem1)

        gwait(rows0, sem0)
        pltpu.sync_copy(rows0, acc_sh.at[dst_v.at[HB - 2]], add=True)
        gwait(rows1, sem1)
        pltpu.sync_copy(rows1, acc_sh.at[dst_v.at[HB - 1]], add=True)

    plsc.subcore_barrier()

    @pl.loop(0, RPS // BLK)
    def _(k):
        pltpu.sync_copy(acc_sh.at[pl.ds(s * RPS + k * BLK, BLK)], rows0)
        pltpu.sync_copy(rows0, out_hbm.at[c, pl.ds(s * RPS + k * BLK, BLK)])


def _mm_body(x_ref, w_ref, o_ref):
    o_ref[...] = jnp.dot(x_ref[...], w_ref[...],
                         preferred_element_type=jnp.float32)


def _scale_body(xw_ref, deg_ref, o_ref):
    deg = deg_ref[0, :, 0] + deg_ref[1, :, 0] + 1.0
    dis = lax.rsqrt(deg)
    o_ref[...] = xw_ref[...] * dis[:, None]


def _final_body(acc_ref, xw2_ref, deg_ref, b1_ref, w2_ref, b2_ref, o_ref):
    deg = deg_ref[0, :, 0] + deg_ref[1, :, 0] + 1.0
    dis = lax.rsqrt(deg)
    t = (acc_ref[0] + acc_ref[1] + xw2_ref[...]) * dis[:, None] + b1_ref[...]
    h = jnp.maximum(t, 0.0)
    o_ref[...] = jnp.dot(h, w2_ref[...],
                         preferred_element_type=jnp.float32) + b2_ref[...]


def kernel(x, edge_index, W1, b1, W2, b2):
    E = edge_index.shape[1]
    n_pad = EPAD - E
    pad_idx = N + (jnp.arange(n_pad, dtype=jnp.int32) % (NP - N))
    src = jnp.concatenate([edge_index[0], pad_idx]).reshape(NW * NBLK, BLK)
    dst = jnp.concatenate([edge_index[1], pad_idx]).reshape(NW * NBLK, BLK)

    x_pad = jnp.pad(x, ((0, NP - N), (0, 0)))
    ones128 = jnp.ones((BLK, D), jnp.float32)
    zeros128 = jnp.zeros((BLK, D), jnp.float32)
    b1r = b1.reshape(1, D)
    b2r = b2.reshape(1, D)

    degs = _deg_kernel(dst, ones128, zeros128)
    xw = pl.pallas_call(
        _mm_body,
        grid=(NP // MB,),
        in_specs=[pl.BlockSpec((MB, D), lambda i: (i, 0)),
                  pl.BlockSpec((D, D), lambda i: (0, 0))],
        out_specs=pl.BlockSpec((MB, D), lambda i: (i, 0)),
        out_shape=jax.ShapeDtypeStruct((NP, D), jnp.float32),
    )(x_pad, W1)

    xw2 = pl.pallas_call(
        _scale_body,
        grid=(NP // MB,),
        in_specs=[pl.BlockSpec((MB, D), lambda i: (i, 0)),
                  pl.BlockSpec((NC, MB, D), lambda i: (0, i, 0))],
        out_specs=pl.BlockSpec((MB, D), lambda i: (i, 0)),
        out_shape=jax.ShapeDtypeStruct((NP, D), jnp.float32),
    )(xw, degs)

    accs = _scatter_kernel(src, dst, xw2)

    out = pl.pallas_call(
        _final_body,
        grid=(NP // MB,),
        in_specs=[pl.BlockSpec((NC, MB, D), lambda i: (0, i, 0)),
                  pl.BlockSpec((MB, D), lambda i: (i, 0)),
                  pl.BlockSpec((NC, MB, D), lambda i: (0, i, 0)),
                  pl.BlockSpec((1, D), lambda i: (0, 0)),
                  pl.BlockSpec((D, D), lambda i: (0, 0)),
                  pl.BlockSpec((1, D), lambda i: (0, 0))],
        out_specs=pl.BlockSpec((MB, D), lambda i: (i, 0)),
        out_shape=jax.ShapeDtypeStruct((NP, D), jnp.float32),
    )(accs, xw2, degs, b1r, W2, b2r)

    return out[:N]


# speedup vs baseline: 41.1517x; 1.3083x over previous
import dataclasses
import functools

import jax
import jax.numpy as jnp
from jax import lax
from jax.experimental import pallas as pl
from jax.experimental.pallas import tpu as pltpu
from jax.experimental.pallas import tpu_sc as plsc

N = 10000
D = 128
NC = 2
NS = 16
NW = NC * NS
BLK = 128
NBLK = 80
HB = NBLK // 2
EPW = BLK * NBLK
EPAD = NW * EPW
NP = 10240
RPS = NP // NS
DEG_W = 16
MB = NP // 8

_mesh = plsc.VectorSubcoreMesh(core_axis_name="c", subcore_axis_name="s")


@functools.partial(
    pl.kernel,
    out_type=jax.ShapeDtypeStruct((NC, NP, DEG_W), jnp.float32),
    mesh=_mesh,
    scratch_types=[
        pltpu.VMEM_SHARED((NS, NP), jnp.float32),
        pltpu.VMEM((NP,), jnp.float32),
        pltpu.VMEM((EPW,), jnp.int32),
        pltpu.VMEM((NS, RPS), jnp.float32),
        pltpu.VMEM((RPS,), jnp.float32),
        pltpu.VMEM((RPS, DEG_W), jnp.float32),
    ],
    compiler_params=dataclasses.replace(pltpu.CompilerParams(),
                                        needs_layout_passes=False),
)
def _deg_kernel(dst_hbm, out_hbm, stage_sh, hist, idx_v, red, redout, rep):
    c = lax.axis_index("c")
    s = lax.axis_index("s")
    wid = s * NC + c

    @pl.loop(0, NP // 16)
    def _(i):
        hist[pl.ds(i * 16, 16)] = jnp.zeros((16,), jnp.float32)

    pltpu.sync_copy(dst_hbm.at[pl.ds(wid * EPW, EPW)], idx_v)
    ones = jnp.ones((16,), jnp.float32)

    @pl.loop(0, NBLK)
    def _(j):
        for kk in range(BLK // 16):
            v = idx_v[pl.ds(j * BLK + kk * 16, 16)]
            plsc.addupdate_scatter(hist, [v], ones)

    pltpu.sync_copy(hist, stage_sh.at[s])
    plsc.subcore_barrier()

    for r in range(NS):
        pltpu.sync_copy(stage_sh.at[r, pl.ds(s * RPS, RPS)], red.at[r])

    @pl.loop(0, RPS // 16)
    def _(t):
        tot = red[0, pl.ds(t * 16, 16)]
        for r in range(1, NS):
            tot = tot + red[r, pl.ds(t * 16, 16)]
        redout[pl.ds(t * 16, 16)] = tot

    @pl.loop(0, RPS)
    def _(j):
        rep[j, :] = plsc.load_gather(redout, [jnp.full((16,), j, jnp.int32)])

    pltpu.sync_copy(rep, out_hbm.at[c, pl.ds(s * RPS, RPS)])


@functools.partial(
    pl.kernel,
    out_type=jax.ShapeDtypeStruct((NC, NP, D), jnp.float32),
    mesh=_mesh,
    scratch_types=[
        pltpu.VMEM_SHARED((NP, D), jnp.float32),
        pltpu.VMEM((HB, BLK), jnp.int32),
        pltpu.VMEM((HB, BLK), jnp.int32),
        pltpu.VMEM((BLK, D), jnp.float32),
        pltpu.VMEM((BLK, D), jnp.float32),
        pltpu.SemaphoreType.DMA,
        pltpu.SemaphoreType.DMA,
    ],
)
def _scatter_kernel(src_hbm, dst_hbm, table_hbm, out_hbm,
                    acc_sh, src_v, dst_v, rows0, rows1, sem0, sem1):
    c = lax.axis_index("c")
    s = lax.axis_index("s")
    wid = s * NC + c
    pltpu.sync_copy(table_hbm.at[pl.ds(N, BLK)], rows0)

    @pl.loop(0, RPS // BLK)
    def _(k):
        pltpu.sync_copy(rows0, acc_sh.at[pl.ds(s * RPS + k * BLK, BLK)])

    plsc.subcore_barrier()

    def gather(j, buf, sem):
        pltpu.make_async_copy(table_hbm.at[src_v.at[j]], buf, sem).start()

    def gwait(buf, sem):
        pltpu.make_async_copy(table_hbm.at[src_v.at[0]], buf, sem).wait()

    @pl.loop(0, 2)
    def _(h):
        base = wid * NBLK + h * HB
        pltpu.sync_copy(src_hbm.at[pl.ds(base, HB)], src_v)
        pltpu.sync_copy(dst_hbm.at[pl.ds(base, HB)], dst_v)

        gather(0, rows0, sem0)
        gather(1, rows1, sem1)

        @pl.loop(0, HB - 2, step=2)
        def _(j):
            gwait(rows0, sem0)
            pltpu.sync_copy(rows0, acc_sh.at[dst_v.at[j]], add=True)
            gather(j + 2, rows0, sem0)
            gwait(rows1, sem1)
            pltpu.sync_copy(rows1, acc_sh.at[dst_v.at[j + 1]], add=True)
            gather(j + 3, rows1, sem1)

        gwait(rows0, sem0)
        pltpu.sync_copy(rows0, acc_sh.at[dst_v.at[HB - 2]], add=True)
        gwait(rows1, sem1)
        pltpu.sync_copy(rows1, acc_sh.at[dst_v.at[HB - 1]], add=True)

    plsc.subcore_barrier()

    @pl.loop(0, RPS // BLK)
    def _(k):
        pltpu.sync_copy(acc_sh.at[pl.ds(s * RPS + k * BLK, BLK)], rows0)
        pltpu.sync_copy(rows0, out_hbm.at[c, pl.ds(s * RPS + k * BLK, BLK)])


def _mm_body(x_ref, w_ref, o_ref):
    o_ref[...] = jnp.dot(x_ref[...], w_ref[...],
                         preferred_element_type=jnp.float32)


def _scale_body(xw_ref, deg_ref, o_ref):
    deg = deg_ref[0, :, 0] + deg_ref[1, :, 0] + 1.0
    dis = lax.rsqrt(deg)
    o_ref[...] = xw_ref[...] * dis[:, None]


def _final_body(acc_ref, xw2_ref, deg_ref, b1_ref, w2_ref, b2_ref, o_ref):
    deg = deg_ref[0, :, 0] + deg_ref[1, :, 0] + 1.0
    dis = lax.rsqrt(deg)
    t = (acc_ref[0] + acc_ref[1] + xw2_ref[...]) * dis[:, None] + b1_ref[...]
    h = jnp.maximum(t, 0.0)
    o_ref[...] = jnp.dot(h, w2_ref[...],
                         preferred_element_type=jnp.float32) + b2_ref[...]


def kernel(x, edge_index, W1, b1, W2, b2):
    E = edge_index.shape[1]
    n_pad = EPAD - E
    pad_idx = N + (jnp.arange(n_pad, dtype=jnp.int32) % (NP - N))
    src = jnp.concatenate([edge_index[0], pad_idx]).reshape(NW * NBLK, BLK)
    dstf = jnp.concatenate([edge_index[1], pad_idx])
    dst = dstf.reshape(NW * NBLK, BLK)

    x_pad = jnp.pad(x, ((0, NP - N), (0, 0)))
    b1r = b1.reshape(1, D)
    b2r = b2.reshape(1, D)

    degs = _deg_kernel(dstf)
    xw = pl.pallas_call(
        _mm_body,
        grid=(NP // MB,),
        in_specs=[pl.BlockSpec((MB, D), lambda i: (i, 0)),
                  pl.BlockSpec((D, D), lambda i: (0, 0))],
        out_specs=pl.BlockSpec((MB, D), lambda i: (i, 0)),
        out_shape=jax.ShapeDtypeStruct((NP, D), jnp.float32),
    )(x_pad, W1)

    xw2 = pl.pallas_call(
        _scale_body,
        grid=(NP // MB,),
        in_specs=[pl.BlockSpec((MB, D), lambda i: (i, 0)),
                  pl.BlockSpec((NC, MB, DEG_W), lambda i: (0, i, 0))],
        out_specs=pl.BlockSpec((MB, D), lambda i: (i, 0)),
        out_shape=jax.ShapeDtypeStruct((NP, D), jnp.float32),
    )(xw, degs)

    accs = _scatter_kernel(src, dst, xw2)

    out = pl.pallas_call(
        _final_body,
        grid=(NP // MB,),
        in_specs=[pl.BlockSpec((NC, MB, D), lambda i: (0, i, 0)),
                  pl.BlockSpec((MB, D), lambda i: (i, 0)),
                  pl.BlockSpec((NC, MB, DEG_W), lambda i: (0, i, 0)),
                  pl.BlockSpec((1, D), lambda i: (0, 0)),
                  pl.BlockSpec((D, D), lambda i: (0, 0)),
                  pl.BlockSpec((1, D), lambda i: (0, 0))],
        out_specs=pl.BlockSpec((MB, D), lambda i: (i, 0)),
        out_shape=jax.ShapeDtypeStruct((NP, D), jnp.float32),
    )(accs, xw2, degs, b1r, W2, b2r)

    return out[:N]
